# Initial kernel scaffold; baseline (speedup 1.0000x reference)
import jax
import jax.numpy as jnp
from jax.experimental import pallas as pl


def kernel(index, output, label, target):
    raise NotImplementedError("write your pallas kernel here")


# speedup vs baseline: 2.0034x; 2.0034x over previous
import functools

import jax
import jax.numpy as jnp
from jax import lax
from jax.experimental import pallas as pl
from jax.experimental.pallas import tpu as pltpu
from jax.experimental.pallas import tpu_sc as plsc

_NUM_EXAMP = 1000000
_NUM_CLASSES = 32
_BATCH = 16384
_LAMBDA_REG = 3.0
_BETA = 0.7

_NC = 2
_NS = 16
_NW = _NC * _NS
_BPW = _BATCH // _NW
_CHUNK = 128
_K = _BPW // _CHUNK

@functools.cache
def _sc_kernels():
    mesh = plsc.VectorSubcoreMesh(core_axis_name="c", subcore_axis_name="s")
    params = pltpu.CompilerParams(use_tc_tiling_on_sc=False)
    scratch = [
        pltpu.VMEM((_K, _CHUNK), jnp.int32),
        pltpu.VMEM((_BPW, _NUM_CLASSES), jnp.float32),
        pltpu.SemaphoreType.DMA,
    ]

    @functools.partial(
        pl.kernel,
        out_type=jax.ShapeDtypeStruct((_BATCH, _NUM_CLASSES), jnp.float32),
        mesh=mesh,
        scratch_types=scratch,
        compiler_params=params,
    )
    def sc_gather(table_hbm, idx_hbm, out_hbm, idx_v, rows_v, sem):
        wid = lax.axis_index("s") * _NC + lax.axis_index("c")
        pltpu.sync_copy(idx_hbm.at[pl.ds(wid * _K, _K)], idx_v)
        handles = []
        for j in range(_K):
            handles.append(
                pltpu.async_copy(
                    table_hbm.at[idx_v.at[j]],
                    rows_v.at[pl.ds(j * _CHUNK, _CHUNK)],
                    sem,
                )
            )
        for h in handles:
            h.wait()
        pltpu.sync_copy(rows_v, out_hbm.at[pl.ds(wid * _BPW, _BPW)])

    @functools.partial(
        pl.kernel,
        out_type=(),
        mesh=mesh,
        scratch_types=scratch,
        compiler_params=params,
    )
    def sc_scatter(tgt_ref_hbm, idx_hbm, rows_hbm, idx_v, rows_v, sem):
        wid = lax.axis_index("s") * _NC + lax.axis_index("c")
        pltpu.sync_copy(idx_hbm.at[pl.ds(wid * _K, _K)], idx_v)
        pltpu.sync_copy(rows_hbm.at[pl.ds(wid * _BPW, _BPW)], rows_v)
        handles = []
        for j in range(_K):
            handles.append(
                pltpu.async_copy(
                    rows_v.at[pl.ds(j * _CHUNK, _CHUNK)],
                    tgt_ref_hbm.at[idx_v.at[j]],
                    sem,
                )
            )
        for h in handles:
            h.wait()

    return sc_gather, sc_scatter


_BB = 1024
_GRID = _BATCH // _BB


def _dense_body(out_ref, lab_ref, avg_ref, newrows_ref, loss_ref, ce_acc, reg_acc):
    i = pl.program_id(0)
    o = out_ref[...]
    m = jnp.max(o, axis=1, keepdims=True)
    e = jnp.exp(o - m)
    s = jnp.sum(e, axis=1, keepdims=True)
    y = jnp.clip(e / s, 0.0001, 1.0 - 0.0001)
    logp = (o - m) - jnp.log(s)
    lab = lab_ref[...]
    onehot = lab == lax.broadcasted_iota(jnp.int32, (_BB, _NUM_CLASSES), 1)
    ce_blk = jnp.sum(jnp.where(onehot, logp, 0.0))
    avg = avg_ref[...]
    dot = jnp.sum(avg * y, axis=1, keepdims=True)
    reg_blk = jnp.sum(jnp.log(1.0 - dot))
    newrows_ref[...] = _BETA * avg + (1.0 - _BETA) * y

    @pl.when(i == 0)
    def _():
        ce_acc[0, 0] = 0.0
        reg_acc[0, 0] = 0.0

    ce_acc[0, 0] += ce_blk
    reg_acc[0, 0] += reg_blk
    inv_b = 1.0 / _BATCH
    loss_ref[0, 0] = -ce_acc[0, 0] * inv_b + _LAMBDA_REG * reg_acc[0, 0] * inv_b


_dense = pl.pallas_call(
    _dense_body,
    grid=(_GRID,),
    in_specs=[
        pl.BlockSpec((_BB, _NUM_CLASSES), lambda i: (i, 0)),
        pl.BlockSpec((_BB, 1), lambda i: (i, 0)),
        pl.BlockSpec((_BB, _NUM_CLASSES), lambda i: (i, 0)),
    ],
    out_specs=[
        pl.BlockSpec((_BB, _NUM_CLASSES), lambda i: (i, 0)),
        pl.BlockSpec(memory_space=pltpu.SMEM),
    ],
    out_shape=[
        jax.ShapeDtypeStruct((_BATCH, _NUM_CLASSES), jnp.float32),
        jax.ShapeDtypeStruct((1, 1), jnp.float32),
    ],
    scratch_shapes=[
        pltpu.SMEM((1, 1), jnp.float32),
        pltpu.SMEM((1, 1), jnp.float32),
    ],
)


def kernel(index, output, label, target):
    sc_gather, sc_scatter = _sc_kernels()
    idx2 = index.astype(jnp.int32).reshape(_BATCH // _CHUNK, _CHUNK)
    y_pred_avg = sc_gather(target, idx2)
    new_rows, loss2 = _dense(output, label.astype(jnp.int32).reshape(_BATCH, 1),
                             y_pred_avg)
    t_ref = jax.new_ref(target)
    sc_scatter(t_ref, idx2, new_rows)
    new_target = jax.freeze(t_ref)
    return loss2[0, 0], new_target
